# baseline (speedup 1.0000x reference)
import functools

import jax
import jax.numpy as jnp
from jax import lax
from jax.experimental import pallas as pl
from jax.experimental.pallas import tpu as pltpu
from jax.experimental.pallas import tpu_sc as plsc

N = 10000
E = 160000
D = 256
H = 512
PE = 64
G = 16

BLK = 1000
NCH = 4
CW = 128
NT = 16
EC = 128
HCH = 40
ECH = 2 * HCH
EPAD = NT * ECH * EC
NTR = 8
RPT = 624
RTAIL = N - NT * RPT


def _enc_body(pe_ref, x_ref, w_ref, be_ref, o_ref):
    lhs = jnp.concatenate([pe_ref[...], x_ref[...]], axis=1)
    o_ref[...] = (jnp.dot(lhs, w_ref[...], preferred_element_type=jnp.float32)
                  + be_ref[...])


def _encoder(pe, x, w, be):
    return pl.pallas_call(
        _enc_body,
        grid=(N // BLK,),
        in_specs=[
            pl.BlockSpec((BLK, PE), lambda i: (i, 0)),
            pl.BlockSpec((BLK, D), lambda i: (i, 0)),
            pl.BlockSpec((PE + D, H), lambda i: (0, 0)),
            pl.BlockSpec((1, H), lambda i: (0, 0)),
        ],
        out_specs=pl.BlockSpec((BLK, H), lambda i: (i, 0)),
        out_shape=jax.ShapeDtypeStruct((N, H), jnp.float32),
    )(pe, x, w, be)


def _mm0_body(h_ref, w_ref, deg_ref, o0, o1, o2, o3):
    dinv = 1.0 / jnp.sqrt(deg_ref[...])
    res = jnp.dot(h_ref[...], w_ref[...],
                  preferred_element_type=jnp.float32) * dinv
    for k, o in enumerate((o0, o1, o2, o3)):
        o[...] = res[:, k * CW:(k + 1) * CW]


def _layer_mm0(h, w, deg):
    return pl.pallas_call(
        _mm0_body,
        grid=(N // BLK,),
        in_specs=[
            pl.BlockSpec((BLK, H), lambda i: (i, 0)),
            pl.BlockSpec((H, H), lambda i: (0, 0)),
            pl.BlockSpec((BLK, 1), lambda i: (i, 0)),
        ],
        out_specs=[pl.BlockSpec((BLK, CW), lambda i: (i, 0))] * NCH,
        out_shape=[jax.ShapeDtypeStruct((N, CW), jnp.float32)] * NCH,
    )(h, w, deg)


def _mm_body(a0, a1, a2, a3, w_ref, deg_ref, bp_ref, o0, o1, o2, o3):
    dinv = 1.0 / jnp.sqrt(deg_ref[...])
    agg = jnp.concatenate([a0[...], a1[...], a2[...], a3[...]], axis=1)
    x = jnp.maximum(agg * dinv + bp_ref[...], 0.0)
    res = jnp.dot(x, w_ref[...], preferred_element_type=jnp.float32) * dinv
    for k, o in enumerate((o0, o1, o2, o3)):
        o[...] = res[:, k * CW:(k + 1) * CW]


def _layer_mm(aggs, w, deg, b_prev):
    return pl.pallas_call(
        _mm_body,
        grid=(N // BLK,),
        in_specs=[pl.BlockSpec((BLK, CW), lambda i: (i, 0))] * NCH + [
            pl.BlockSpec((H, H), lambda i: (0, 0)),
            pl.BlockSpec((BLK, 1), lambda i: (i, 0)),
            pl.BlockSpec((1, H), lambda i: (0, 0)),
        ],
        out_specs=[pl.BlockSpec((BLK, CW), lambda i: (i, 0))] * NCH,
        out_shape=[jax.ShapeDtypeStruct((N, CW), jnp.float32)] * NCH,
    )(*aggs, w, deg, b_prev)


def _sc_pass(ms_hbm, out_hbm, src_hbm, dst_hbm, acc, srcl, dstl,
             buf0, buf1, sem0, sem1, s):
    pltpu.sync_copy(ms_hbm.at[pl.ds(s * RPT, RPT)], acc.at[pl.ds(s * RPT, RPT)])

    @pl.when(s == 0)
    def _():
        pltpu.sync_copy(ms_hbm.at[pl.ds(NT * RPT, RTAIL)],
                        acc.at[pl.ds(NT * RPT, RTAIL)])

    plsc.subcore_barrier()

    for he in range(2):
        pltpu.sync_copy(src_hbm.at[s].at[he], srcl)
        pltpu.sync_copy(dst_hbm.at[s].at[he], dstl)
        pltpu.async_copy(ms_hbm.at[srcl.at[0]], buf0, sem0)

        def pair(i, carry):
            j0 = i * 2
            pltpu.make_async_copy(ms_hbm.at[srcl.at[j0]], buf0, sem0).wait()
            pltpu.async_copy(ms_hbm.at[srcl.at[j0 + 1]], buf1, sem1)
            pltpu.sync_copy(buf0, acc.at[dstl.at[j0]], add=True)
            pltpu.make_async_copy(ms_hbm.at[srcl.at[j0 + 1]], buf1, sem1).wait()

            @pl.when(j0 + 2 < HCH)
            def _():
                pltpu.async_copy(ms_hbm.at[srcl.at[j0 + 2]], buf0, sem0)

            pltpu.sync_copy(buf1, acc.at[dstl.at[j0 + 1]], add=True)
            return carry

        lax.fori_loop(0, HCH // 2, pair, 0)
    plsc.subcore_barrier()
    pltpu.sync_copy(acc.at[pl.ds(s * RPT, RPT)], out_hbm.at[pl.ds(s * RPT, RPT)])

    @pl.when(s == 0)
    def _():
        pltpu.sync_copy(acc.at[pl.ds(NT * RPT, RTAIL)],
                        out_hbm.at[pl.ds(NT * RPT, RTAIL)])

    plsc.subcore_barrier()


def _scatter_body(ms0, ms1, ms2, ms3, src_hbm, dst_hbm,
                  out0, out1, out2, out3, acc, srcl, dstl, buf0, buf1,
                  sem0, sem1):
    c = lax.axis_index("c")
    s = lax.axis_index("s")
    mss = (ms0, ms1, ms2, ms3)
    outs = (out0, out1, out2, out3)
    for half in range(2):
        @pl.when(c == 0)
        def _():
            _sc_pass(mss[half], outs[half], src_hbm, dst_hbm, acc, srcl, dstl,
                     buf0, buf1, sem0, sem1, s)

        @pl.when(c == 1)
        def _():
            _sc_pass(mss[2 + half], outs[2 + half], src_hbm, dst_hbm, acc,
                     srcl, dstl, buf0, buf1, sem0, sem1, s)


def _make_scatter():
    mesh = plsc.VectorSubcoreMesh(core_axis_name="c", subcore_axis_name="s")
    return pl.kernel(
        _scatter_body,
        out_type=[jax.ShapeDtypeStruct((N, CW), jnp.float32)] * NCH,
        mesh=mesh,
        scratch_types=[
            pltpu.VMEM_SHARED((N + NTR, CW), jnp.float32),
            pltpu.VMEM((HCH, EC), jnp.int32),
            pltpu.VMEM((HCH, EC), jnp.int32),
            pltpu.VMEM((EC, CW), jnp.float32),
            pltpu.VMEM((EC, CW), jnp.float32),
            pltpu.SemaphoreType.DMA,
            pltpu.SemaphoreType.DMA,
        ],
    )


def kernel(x, edge_index, batch, part_ids, embeddings, W_enc, b_enc,
           W_g0, b_g0, W_g1, b_g1, W_g2, b_g2,
           W_a1, b_a1, W_a2, b_a2, W_v1, b_v1, W_v2, b_v2):
    n = x.shape[0]
    src0, dst0 = edge_index[0], edge_index[1]
    npad = EPAD - E
    src_r = jnp.concatenate(
        [src0, jnp.zeros((npad,), src0.dtype)]).reshape(NT, 2, HCH, EC)
    dst_r = jnp.concatenate(
        [dst0, N + (jnp.arange(npad, dtype=dst0.dtype) % NTR)]
    ).reshape(NT, 2, HCH, EC)

    deg = jax.ops.segment_sum(jnp.ones_like(src0, dtype=jnp.float32), dst0,
                              num_segments=n) + 1.0
    deg2 = deg[:, None]

    pe = jnp.take(embeddings, part_ids, axis=0)
    h = _encoder(pe, x, W_enc, b_enc[None, :])

    scat = _make_scatter()
    ms = _layer_mm0(h, W_g0, deg2)
    aggs = scat(*ms, src_r, dst_r)
    for W, b_prev in ((W_g1, b_g0), (W_g2, b_g1)):
        ms = _layer_mm(aggs, W, deg2, b_prev[None, :])
        aggs = scat(*ms, src_r, dst_r)

    dinv = 1.0 / jnp.sqrt(deg)
    agg = jnp.concatenate(aggs, axis=1)
    h = jax.nn.relu(agg * dinv[:, None] + b_g2)

    adv = jax.nn.relu(h @ W_a1 + b_a1) @ W_a2 + b_a2
    cnt = jnp.maximum(jax.ops.segment_sum(jnp.ones((n,), h.dtype), batch,
                                          num_segments=G), 1.0)
    adv_mean = (jax.ops.segment_sum(adv, batch, num_segments=G) / cnt[:, None])[batch]
    value_x = jax.ops.segment_sum(h, batch, num_segments=G) / cnt[:, None]
    value = (jax.nn.relu(value_x @ W_v1 + b_v1) @ W_v2 + b_v2)[batch]
    return value + adv - adv_mean


# speedup vs baseline: 1.0934x; 1.0934x over previous
import functools

import jax
import jax.numpy as jnp
from jax import lax
from jax.experimental import pallas as pl
from jax.experimental.pallas import tpu as pltpu
from jax.experimental.pallas import tpu_sc as plsc

N = 10000
E = 160000
D = 256
H = 512
PE = 64
G = 16

BLK = 1000
NCH = 4
CW = 128
NT = 16
EC = 128
HCH = 40
ECH = 2 * HCH
EPAD = NT * ECH * EC
NTR = 16
RPT = 624
RTAIL = N - NT * RPT


def _enc_body(pe_ref, x_ref, w_ref, be_ref, o_ref):
    lhs = jnp.concatenate([pe_ref[...], x_ref[...]], axis=1)
    o_ref[...] = (jnp.dot(lhs, w_ref[...], preferred_element_type=jnp.float32)
                  + be_ref[...])


def _encoder(pe, x, w, be):
    return pl.pallas_call(
        _enc_body,
        grid=(N // BLK,),
        in_specs=[
            pl.BlockSpec((BLK, PE), lambda i: (i, 0)),
            pl.BlockSpec((BLK, D), lambda i: (i, 0)),
            pl.BlockSpec((PE + D, H), lambda i: (0, 0)),
            pl.BlockSpec((1, H), lambda i: (0, 0)),
        ],
        out_specs=pl.BlockSpec((BLK, H), lambda i: (i, 0)),
        out_shape=jax.ShapeDtypeStruct((N, H), jnp.float32),
    )(pe, x, w, be)


def _mm0_body(h_ref, w_ref, deg_ref, o0, o1, o2, o3):
    dinv = 1.0 / jnp.sqrt(deg_ref[...])
    res = jnp.dot(h_ref[...], w_ref[...],
                  preferred_element_type=jnp.float32) * dinv
    for k, o in enumerate((o0, o1, o2, o3)):
        o[...] = res[:, k * CW:(k + 1) * CW]


def _layer_mm0(h, w, deg):
    return pl.pallas_call(
        _mm0_body,
        grid=(N // BLK,),
        in_specs=[
            pl.BlockSpec((BLK, H), lambda i: (i, 0)),
            pl.BlockSpec((H, H), lambda i: (0, 0)),
            pl.BlockSpec((BLK, 1), lambda i: (i, 0)),
        ],
        out_specs=[pl.BlockSpec((BLK, CW), lambda i: (i, 0))] * NCH,
        out_shape=[jax.ShapeDtypeStruct((N, CW), jnp.float32)] * NCH,
    )(h, w, deg)


def _mm_body(a0, a1, a2, a3, w_ref, deg_ref, bp_ref, o0, o1, o2, o3):
    dinv = 1.0 / jnp.sqrt(deg_ref[...])
    agg = jnp.concatenate([a0[...], a1[...], a2[...], a3[...]], axis=1)
    x = jnp.maximum(agg * dinv + bp_ref[...], 0.0)
    res = jnp.dot(x, w_ref[...], preferred_element_type=jnp.float32) * dinv
    for k, o in enumerate((o0, o1, o2, o3)):
        o[...] = res[:, k * CW:(k + 1) * CW]


def _layer_mm(aggs, w, deg, b_prev):
    return pl.pallas_call(
        _mm_body,
        grid=(N // BLK,),
        in_specs=[pl.BlockSpec((BLK, CW), lambda i: (i, 0))] * NCH + [
            pl.BlockSpec((H, H), lambda i: (0, 0)),
            pl.BlockSpec((BLK, 1), lambda i: (i, 0)),
            pl.BlockSpec((1, H), lambda i: (0, 0)),
        ],
        out_specs=[pl.BlockSpec((BLK, CW), lambda i: (i, 0))] * NCH,
        out_shape=[jax.ShapeDtypeStruct((N, CW), jnp.float32)] * NCH,
    )(*aggs, w, deg, b_prev)


def _sc_pass(ms_hbm, out_hbm, src_hbm, dst_hbm, acc, srcl, dstl,
             buf0, buf1, sem0, sem1, s):
    pltpu.sync_copy(ms_hbm.at[pl.ds(s * RPT, RPT)], acc.at[pl.ds(s * RPT, RPT)])

    @pl.when(s == 0)
    def _():
        pltpu.sync_copy(ms_hbm.at[pl.ds(NT * RPT, RTAIL)],
                        acc.at[pl.ds(NT * RPT, RTAIL)])

    plsc.subcore_barrier()

    for he in range(2):
        pltpu.sync_copy(src_hbm.at[s].at[he], srcl)
        pltpu.sync_copy(dst_hbm.at[s].at[he], dstl)
        pltpu.async_copy(ms_hbm.at[srcl.at[0]], buf0, sem0)

        def pair(i, carry):
            j0 = i * 2
            pltpu.make_async_copy(ms_hbm.at[srcl.at[j0]], buf0, sem0).wait()
            pltpu.async_copy(ms_hbm.at[srcl.at[j0 + 1]], buf1, sem1)
            pltpu.sync_copy(buf0, acc.at[dstl.at[j0]], add=True)
            pltpu.make_async_copy(ms_hbm.at[srcl.at[j0 + 1]], buf1, sem1).wait()

            @pl.when(j0 + 2 < HCH)
            def _():
                pltpu.async_copy(ms_hbm.at[srcl.at[j0 + 2]], buf0, sem0)

            pltpu.sync_copy(buf1, acc.at[dstl.at[j0 + 1]], add=True)
            return carry

        lax.fori_loop(0, HCH // 2, pair, 0)
    plsc.subcore_barrier()
    pltpu.sync_copy(acc.at[pl.ds(s * RPT, RPT)], out_hbm.at[pl.ds(s * RPT, RPT)])

    @pl.when(s == 0)
    def _():
        pltpu.sync_copy(acc.at[pl.ds(NT * RPT, RTAIL)],
                        out_hbm.at[pl.ds(NT * RPT, RTAIL)])

    plsc.subcore_barrier()


def _scatter_body(ms0, ms1, ms2, ms3, src_hbm, dst_hbm,
                  out0, out1, out2, out3, acc, srcl, dstl, buf0, buf1,
                  sem0, sem1):
    c = lax.axis_index("c")
    s = lax.axis_index("s")
    mss = (ms0, ms1, ms2, ms3)
    outs = (out0, out1, out2, out3)
    for half in range(2):
        @pl.when(c == 0)
        def _():
            _sc_pass(mss[half], outs[half], src_hbm, dst_hbm, acc, srcl, dstl,
                     buf0, buf1, sem0, sem1, s)

        @pl.when(c == 1)
        def _():
            _sc_pass(mss[2 + half], outs[2 + half], src_hbm, dst_hbm, acc,
                     srcl, dstl, buf0, buf1, sem0, sem1, s)


def _make_scatter():
    mesh = plsc.VectorSubcoreMesh(core_axis_name="c", subcore_axis_name="s")
    return pl.kernel(
        _scatter_body,
        out_type=[jax.ShapeDtypeStruct((N, CW), jnp.float32)] * NCH,
        mesh=mesh,
        scratch_types=[
            pltpu.VMEM_SHARED((N + NTR, CW), jnp.float32),
            pltpu.VMEM((HCH, EC), jnp.int32),
            pltpu.VMEM((HCH, EC), jnp.int32),
            pltpu.VMEM((EC, CW), jnp.float32),
            pltpu.VMEM((EC, CW), jnp.float32),
            pltpu.SemaphoreType.DMA,
            pltpu.SemaphoreType.DMA,
        ],
    )


def kernel(x, edge_index, batch, part_ids, embeddings, W_enc, b_enc,
           W_g0, b_g0, W_g1, b_g1, W_g2, b_g2,
           W_a1, b_a1, W_a2, b_a2, W_v1, b_v1, W_v2, b_v2):
    n = x.shape[0]
    src0, dst0 = edge_index[0], edge_index[1]
    ppt = (EPAD - E) // NT
    src_r = jnp.concatenate(
        [src0.reshape(NT, E // NT),
         jnp.zeros((NT, ppt), src0.dtype)], axis=1).reshape(NT, 2, HCH, EC)
    dst_r = jnp.concatenate(
        [dst0.reshape(NT, E // NT),
         jnp.broadcast_to(N + jnp.arange(NT, dtype=dst0.dtype)[:, None],
                          (NT, ppt))], axis=1).reshape(NT, 2, HCH, EC)

    deg = jax.ops.segment_sum(jnp.ones_like(src0, dtype=jnp.float32), dst0,
                              num_segments=n) + 1.0
    deg2 = deg[:, None]

    pe = jnp.take(embeddings, part_ids, axis=0)
    h = _encoder(pe, x, W_enc, b_enc[None, :])

    scat = _make_scatter()
    ms = _layer_mm0(h, W_g0, deg2)
    aggs = scat(*ms, src_r, dst_r)
    for W, b_prev in ((W_g1, b_g0), (W_g2, b_g1)):
        ms = _layer_mm(aggs, W, deg2, b_prev[None, :])
        aggs = scat(*ms, src_r, dst_r)

    dinv = 1.0 / jnp.sqrt(deg)
    agg = jnp.concatenate(aggs, axis=1)
    h = jax.nn.relu(agg * dinv[:, None] + b_g2)

    adv = jax.nn.relu(h @ W_a1 + b_a1) @ W_a2 + b_a2
    cnt = jnp.maximum(jax.ops.segment_sum(jnp.ones((n,), h.dtype), batch,
                                          num_segments=G), 1.0)
    adv_mean = (jax.ops.segment_sum(adv, batch, num_segments=G) / cnt[:, None])[batch]
    value_x = jax.ops.segment_sum(h, batch, num_segments=G) / cnt[:, None]
    value = (jax.nn.relu(value_x @ W_v1 + b_v1) @ W_v2 + b_v2)[batch]
    return value + adv - adv_mean


# speedup vs baseline: 1.5156x; 1.3861x over previous
import functools

import jax
import jax.numpy as jnp
from jax import lax
from jax.experimental import pallas as pl
from jax.experimental.pallas import tpu as pltpu
from jax.experimental.pallas import tpu_sc as plsc

N = 10000
E = 160000
D = 256
H = 512
PE = 64
G = 16

BLK = 1000
NCH = 4
CW = 128
NT = 16
EC = 80
ECH = E // NT // EC
EPT = E // NT
RPT = 624
RTAIL = N - NT * RPT
NP = 10240
PPT = NP // NT


def _prep_body(dst_hbm, pid_hbm, emb_hbm, deg_hbm, pe_hbm,
               degloc, dstl, red, pidl, pebuf, sem):
    c = lax.axis_index("c")
    s = lax.axis_index("s")

    @pl.when(c == 0)
    def _():
        zeros16 = jnp.zeros((16,), jnp.float32)

        def z(k, carry):
            degloc[pl.ds(k * 16, 16)] = zeros16
            return carry

        lax.fori_loop(0, NP // 16, z, 0)
        pltpu.sync_copy(dst_hbm.at[s], dstl)
        ones16 = jnp.ones((16,), jnp.float32)

        def cnt(k, carry):
            dv = dstl[pl.ds(k * 16, 16)]
            plsc.addupdate_scatter(degloc, [dv], ones16)
            return carry

        lax.fori_loop(0, EPT // 16, cnt, 0)
        plsc.subcore_barrier()
        pltpu.sync_copy(degloc, red.at[s])
        plsc.subcore_barrier()
        pltpu.sync_copy(red.at[0].at[pl.ds(s * PPT, PPT)], degloc.at[pl.ds(0, PPT)])

        def addk(k, carry):
            pltpu.sync_copy(red.at[k].at[pl.ds(s * PPT, PPT)],
                            degloc.at[pl.ds(PPT, PPT)])

            def va(v, carry2):
                degloc[pl.ds(v * 16, 16)] = (degloc[pl.ds(v * 16, 16)]
                                             + degloc[pl.ds(PPT + v * 16, 16)])
                return carry2

            lax.fori_loop(0, PPT // 16, va, 0)
            return carry

        lax.fori_loop(1, NT, addk, 0)
        pltpu.sync_copy(degloc.at[pl.ds(0, PPT)], deg_hbm.at[pl.ds(s * PPT, PPT)])

    @pl.when(c == 1)
    def _():
        pltpu.sync_copy(pid_hbm.at[s], pidl)
        for g in range(PPT // EC):
            pltpu.async_copy(emb_hbm.at[pidl.at[g]], pebuf, sem).wait()
            pltpu.sync_copy(pebuf, pe_hbm.at[pl.ds(s * PPT + g * EC, EC)])


def _make_prep():
    mesh = plsc.VectorSubcoreMesh(core_axis_name="c", subcore_axis_name="s")
    return pl.kernel(
        _prep_body,
        out_type=[jax.ShapeDtypeStruct((NP,), jnp.float32),
                  jax.ShapeDtypeStruct((NP, PE), jnp.float32)],
        mesh=mesh,
        scratch_types=[
            pltpu.VMEM((NP,), jnp.float32),
            pltpu.VMEM((EPT,), jnp.int32),
            pltpu.VMEM_SHARED((NT, NP), jnp.float32),
            pltpu.VMEM((PPT // EC, EC), jnp.int32),
            pltpu.VMEM((EC, PE), jnp.float32),
            pltpu.SemaphoreType.DMA,
        ],
    )


def _enc_body(pe_ref, x_ref, w_ref, be_ref, o_ref):
    lhs = jnp.concatenate([pe_ref[...], x_ref[...]], axis=1)
    o_ref[...] = (jnp.dot(lhs, w_ref[...], preferred_element_type=jnp.float32)
                  + be_ref[...])


def _encoder(pe, x, w, be):
    return pl.pallas_call(
        _enc_body,
        grid=(N // BLK,),
        in_specs=[
            pl.BlockSpec((BLK, PE), lambda i: (i, 0)),
            pl.BlockSpec((BLK, D), lambda i: (i, 0)),
            pl.BlockSpec((PE + D, H), lambda i: (0, 0)),
            pl.BlockSpec((1, H), lambda i: (0, 0)),
        ],
        out_specs=pl.BlockSpec((BLK, H), lambda i: (i, 0)),
        out_shape=jax.ShapeDtypeStruct((N, H), jnp.float32),
    )(pe, x, w, be)


def _mm0_body(h_ref, w_ref, deg_ref, o0, o1, o2, o3):
    dinv = 1.0 / jnp.sqrt(deg_ref[...] + 1.0)
    res = jnp.dot(h_ref[...], w_ref[...],
                  preferred_element_type=jnp.float32) * dinv
    for k, o in enumerate((o0, o1, o2, o3)):
        o[...] = res[:, k * CW:(k + 1) * CW]


def _layer_mm0(h, w, deg):
    return pl.pallas_call(
        _mm0_body,
        grid=(N // BLK,),
        in_specs=[
            pl.BlockSpec((BLK, H), lambda i: (i, 0)),
            pl.BlockSpec((H, H), lambda i: (0, 0)),
            pl.BlockSpec((BLK, 1), lambda i: (i, 0)),
        ],
        out_specs=[pl.BlockSpec((BLK, CW), lambda i: (i, 0))] * NCH,
        out_shape=[jax.ShapeDtypeStruct((N, CW), jnp.float32)] * NCH,
    )(h, w, deg)


def _mm_body(a0, a1, a2, a3, w_ref, deg_ref, bp_ref, o0, o1, o2, o3):
    dinv = 1.0 / jnp.sqrt(deg_ref[...] + 1.0)
    agg = jnp.concatenate([a0[...], a1[...], a2[...], a3[...]], axis=1)
    x = jnp.maximum(agg * dinv + bp_ref[...], 0.0)
    res = jnp.dot(x, w_ref[...], preferred_element_type=jnp.float32) * dinv
    for k, o in enumerate((o0, o1, o2, o3)):
        o[...] = res[:, k * CW:(k + 1) * CW]


def _layer_mm(aggs, w, deg, b_prev):
    return pl.pallas_call(
        _mm_body,
        grid=(N // BLK,),
        in_specs=[pl.BlockSpec((BLK, CW), lambda i: (i, 0))] * NCH + [
            pl.BlockSpec((H, H), lambda i: (0, 0)),
            pl.BlockSpec((BLK, 1), lambda i: (i, 0)),
            pl.BlockSpec((1, H), lambda i: (0, 0)),
        ],
        out_specs=[pl.BlockSpec((BLK, CW), lambda i: (i, 0))] * NCH,
        out_shape=[jax.ShapeDtypeStruct((N, CW), jnp.float32)] * NCH,
    )(*aggs, w, deg, b_prev)


def _sc_pass(ms_hbm, out_hbm, acc, srcl, dstl, buf, sem, s):
    pltpu.sync_copy(ms_hbm.at[pl.ds(s * RPT, RPT)], acc.at[pl.ds(s * RPT, RPT)])

    @pl.when(s == 0)
    def _():
        pltpu.sync_copy(ms_hbm.at[pl.ds(NT * RPT, RTAIL)],
                        acc.at[pl.ds(NT * RPT, RTAIL)])

    plsc.subcore_barrier()

    def chunk(j, carry):
        pltpu.async_copy(ms_hbm.at[srcl.at[j]], buf, sem).wait()
        pltpu.sync_copy(buf, acc.at[dstl.at[j]], add=True)
        return carry

    lax.fori_loop(0, ECH, chunk, 0)
    plsc.subcore_barrier()
    pltpu.sync_copy(acc.at[pl.ds(s * RPT, RPT)], out_hbm.at[pl.ds(s * RPT, RPT)])

    @pl.when(s == 0)
    def _():
        pltpu.sync_copy(acc.at[pl.ds(NT * RPT, RTAIL)],
                        out_hbm.at[pl.ds(NT * RPT, RTAIL)])

    plsc.subcore_barrier()


def _scatter_body(ms0, ms1, ms2, ms3, src_hbm, dst_hbm,
                  out0, out1, out2, out3, acc, srcl, dstl, buf, sem):
    c = lax.axis_index("c")
    s = lax.axis_index("s")
    pltpu.sync_copy(src_hbm.at[s], srcl)
    pltpu.sync_copy(dst_hbm.at[s], dstl)
    mss = (ms0, ms1, ms2, ms3)
    outs = (out0, out1, out2, out3)
    for half in range(2):
        @pl.when(c == 0)
        def _():
            _sc_pass(mss[half], outs[half], acc, srcl, dstl, buf, sem, s)

        @pl.when(c == 1)
        def _():
            _sc_pass(mss[2 + half], outs[2 + half], acc, srcl, dstl,
                     buf, sem, s)


def _make_scatter():
    mesh = plsc.VectorSubcoreMesh(core_axis_name="c", subcore_axis_name="s")
    return pl.kernel(
        _scatter_body,
        out_type=[jax.ShapeDtypeStruct((N, CW), jnp.float32)] * NCH,
        mesh=mesh,
        scratch_types=[
            pltpu.VMEM_SHARED((N, CW), jnp.float32),
            pltpu.VMEM((ECH, EC), jnp.int32),
            pltpu.VMEM((ECH, EC), jnp.int32),
            pltpu.VMEM((EC, CW), jnp.float32),
            pltpu.SemaphoreType.DMA,
        ],
    )


def _h1_body(a0, a1, a2, a3, deg_ref, bg2_ref, wa1_ref, ba1_ref,
             wa2_ref, ba2_ref, bt_ref, adv_ref, sh_ref, sa_ref,
             sh_acc, sa_acc):
    i = pl.program_id(0)
    dinv = 1.0 / jnp.sqrt(deg_ref[...] + 1.0)
    agg = jnp.concatenate([a0[...], a1[...], a2[...], a3[...]], axis=1)
    h = jnp.maximum(agg * dinv + bg2_ref[...], 0.0)
    z1 = jnp.maximum(jnp.dot(h, wa1_ref[...],
                             preferred_element_type=jnp.float32)
                     + ba1_ref[...], 0.0)
    adv = jnp.dot(z1, wa2_ref[...],
                  preferred_element_type=jnp.float32) + ba2_ref[...]
    adv_ref[...] = adv
    ob = (bt_ref[...] == lax.broadcasted_iota(jnp.int32, (BLK, G), 1)
          ).astype(jnp.float32)

    @pl.when(i == 0)
    def _():
        sh_acc[...] = jnp.zeros_like(sh_acc)
        sa_acc[...] = jnp.zeros_like(sa_acc)

    sh_acc[...] += lax.dot_general(ob, h, (((0,), (0,)), ((), ())),
                                   preferred_element_type=jnp.float32)
    adv_cnt = jnp.concatenate(
        [adv, jnp.ones((BLK, 1), jnp.float32)], axis=1)
    sa_acc[...] += lax.dot_general(ob, adv_cnt, (((0,), (0,)), ((), ())),
                                   preferred_element_type=jnp.float32)

    @pl.when(i == pl.num_programs(0) - 1)
    def _():
        sh_ref[...] = sh_acc[...]
        sa_ref[...] = sa_acc[...]


def _heads1(aggs, deg, b_g2, W_a1, b_a1, W_a2, b_a2, bt):
    return pl.pallas_call(
        _h1_body,
        grid=(N // BLK,),
        in_specs=[pl.BlockSpec((BLK, CW), lambda i: (i, 0))] * NCH + [
            pl.BlockSpec((BLK, 1), lambda i: (i, 0)),
            pl.BlockSpec((1, H), lambda i: (0, 0)),
            pl.BlockSpec((H, H), lambda i: (0, 0)),
            pl.BlockSpec((1, H), lambda i: (0, 0)),
            pl.BlockSpec((H, 1), lambda i: (0, 0)),
            pl.BlockSpec((1, 1), lambda i: (0, 0)),
            pl.BlockSpec((BLK, 1), lambda i: (i, 0)),
        ],
        out_specs=[
            pl.BlockSpec((BLK, 1), lambda i: (i, 0)),
            pl.BlockSpec((G, H), lambda i: (0, 0)),
            pl.BlockSpec((G, 2), lambda i: (0, 0)),
        ],
        out_shape=[
            jax.ShapeDtypeStruct((N, 1), jnp.float32),
            jax.ShapeDtypeStruct((G, H), jnp.float32),
            jax.ShapeDtypeStruct((G, 2), jnp.float32),
        ],
        scratch_shapes=[pltpu.VMEM((G, H), jnp.float32),
                        pltpu.VMEM((G, 2), jnp.float32)],
    )(*aggs, deg, b_g2, W_a1, b_a1, W_a2, b_a2, bt)


def _h2_body(sh_ref, sa_ref, wv1_ref, bv1_ref, wv2_ref, bv2_ref,
             adv_ref, bt_ref, q_ref):
    cnt = jnp.maximum(sa_ref[:, 1:2], 1.0)
    vx = sh_ref[...] / cnt
    z1 = jnp.maximum(jnp.dot(vx, wv1_ref[...],
                             preferred_element_type=jnp.float32)
                     + bv1_ref[...], 0.0)
    value = jnp.dot(z1, wv2_ref[...],
                    preferred_element_type=jnp.float32) + bv2_ref[...]
    corr = value - sa_ref[:, 0:1] / cnt
    ob = (bt_ref[...] == lax.broadcasted_iota(jnp.int32, (BLK, G), 1)
          ).astype(jnp.float32)
    q_ref[...] = adv_ref[...] + jnp.dot(ob, corr,
                                        preferred_element_type=jnp.float32)


def _heads2(sh, sa, W_v1, b_v1, W_v2, b_v2, adv, bt):
    return pl.pallas_call(
        _h2_body,
        grid=(N // BLK,),
        in_specs=[
            pl.BlockSpec((G, H), lambda i: (0, 0)),
            pl.BlockSpec((G, 2), lambda i: (0, 0)),
            pl.BlockSpec((H, H), lambda i: (0, 0)),
            pl.BlockSpec((1, H), lambda i: (0, 0)),
            pl.BlockSpec((H, 1), lambda i: (0, 0)),
            pl.BlockSpec((1, 1), lambda i: (0, 0)),
            pl.BlockSpec((BLK, 1), lambda i: (i, 0)),
            pl.BlockSpec((BLK, 1), lambda i: (i, 0)),
        ],
        out_specs=pl.BlockSpec((BLK, 1), lambda i: (i, 0)),
        out_shape=jax.ShapeDtypeStruct((N, 1), jnp.float32),
    )(sh, sa, W_v1, b_v1, W_v2, b_v2, adv, bt)


def kernel(x, edge_index, batch, part_ids, embeddings, W_enc, b_enc,
           W_g0, b_g0, W_g1, b_g1, W_g2, b_g2,
           W_a1, b_a1, W_a2, b_a2, W_v1, b_v1, W_v2, b_v2):
    src0, dst0 = edge_index[0], edge_index[1]
    src_r = src0.reshape(NT, ECH, EC)
    dst_r = dst0.reshape(NT, ECH, EC)

    deg = jax.ops.segment_sum(jnp.ones_like(src0, dtype=jnp.float32), dst0,
                              num_segments=N)[:, None]
    pe = jnp.take(embeddings, part_ids, axis=0)

    h = _encoder(pe, x, W_enc, b_enc[None, :])

    scat = _make_scatter()
    ms = _layer_mm0(h, W_g0, deg)
    aggs = scat(*ms, src_r, dst_r)
    for W, b_prev in ((W_g1, b_g0), (W_g2, b_g1)):
        ms = _layer_mm(aggs, W, deg, b_prev[None, :])
        aggs = scat(*ms, src_r, dst_r)

    bt = batch.astype(jnp.int32)[:, None]
    adv, sh, sa = _heads1(aggs, deg, b_g2[None, :], W_a1, b_a1[None, :],
                          W_a2, b_a2[None, :], bt)
    return _heads2(sh, sa, W_v1, b_v1[None, :], W_v2, b_v2[None, :], adv, bt)


# speedup vs baseline: 1.5160x; 1.0002x over previous
import functools

import jax
import jax.numpy as jnp
from jax import lax
from jax.experimental import pallas as pl
from jax.experimental.pallas import tpu as pltpu
from jax.experimental.pallas import tpu_sc as plsc

N = 10000
E = 160000
D = 256
H = 512
PE = 64
G = 16

BLK = 1000
NCH = 4
CW = 128
NT = 16
EC = 80
ECH = E // NT // EC
RPT = 624
RTAIL = N - NT * RPT


def _enc_body(pe_ref, x_ref, w_ref, be_ref, o_ref):
    lhs = jnp.concatenate([pe_ref[...], x_ref[...]], axis=1)
    o_ref[...] = (jnp.dot(lhs, w_ref[...], preferred_element_type=jnp.float32)
                  + be_ref[...])


def _encoder(pe, x, w, be):
    return pl.pallas_call(
        _enc_body,
        grid=(N // BLK,),
        in_specs=[
            pl.BlockSpec((BLK, PE), lambda i: (i, 0)),
            pl.BlockSpec((BLK, D), lambda i: (i, 0)),
            pl.BlockSpec((PE + D, H), lambda i: (0, 0)),
            pl.BlockSpec((1, H), lambda i: (0, 0)),
        ],
        out_specs=pl.BlockSpec((BLK, H), lambda i: (i, 0)),
        out_shape=jax.ShapeDtypeStruct((N, H), jnp.float32),
    )(pe, x, w, be)


def _mm0_body(h_ref, w_ref, deg_ref, o0, o1, o2, o3):
    dinv = 1.0 / jnp.sqrt(deg_ref[...] + 1.0)
    res = jnp.dot(h_ref[...], w_ref[...],
                  preferred_element_type=jnp.float32) * dinv
    for k, o in enumerate((o0, o1, o2, o3)):
        o[...] = res[:, k * CW:(k + 1) * CW]


def _layer_mm0(h, w, deg):
    return pl.pallas_call(
        _mm0_body,
        grid=(N // BLK,),
        in_specs=[
            pl.BlockSpec((BLK, H), lambda i: (i, 0)),
            pl.BlockSpec((H, H), lambda i: (0, 0)),
            pl.BlockSpec((BLK, 1), lambda i: (i, 0)),
        ],
        out_specs=[pl.BlockSpec((BLK, CW), lambda i: (i, 0))] * NCH,
        out_shape=[jax.ShapeDtypeStruct((N, CW), jnp.float32)] * NCH,
    )(h, w, deg)


def _mm_body(a0, a1, a2, a3, w_ref, deg_ref, bp_ref, o0, o1, o2, o3):
    dinv = 1.0 / jnp.sqrt(deg_ref[...] + 1.0)
    agg = jnp.concatenate([a0[...], a1[...], a2[...], a3[...]], axis=1)
    x = jnp.maximum(agg * dinv + bp_ref[...], 0.0)
    res = jnp.dot(x, w_ref[...], preferred_element_type=jnp.float32) * dinv
    for k, o in enumerate((o0, o1, o2, o3)):
        o[...] = res[:, k * CW:(k + 1) * CW]


def _layer_mm(aggs, w, deg, b_prev):
    return pl.pallas_call(
        _mm_body,
        grid=(N // BLK,),
        in_specs=[pl.BlockSpec((BLK, CW), lambda i: (i, 0))] * NCH + [
            pl.BlockSpec((H, H), lambda i: (0, 0)),
            pl.BlockSpec((BLK, 1), lambda i: (i, 0)),
            pl.BlockSpec((1, H), lambda i: (0, 0)),
        ],
        out_specs=[pl.BlockSpec((BLK, CW), lambda i: (i, 0))] * NCH,
        out_shape=[jax.ShapeDtypeStruct((N, CW), jnp.float32)] * NCH,
    )(*aggs, w, deg, b_prev)


def _sc_pass(ms_hbm, out_hbm, acc, srcl, dstl, buf, sem, s):
    pltpu.sync_copy(ms_hbm.at[pl.ds(s * RPT, RPT)], acc.at[pl.ds(s * RPT, RPT)])

    @pl.when(s == 0)
    def _():
        pltpu.sync_copy(ms_hbm.at[pl.ds(NT * RPT, RTAIL)],
                        acc.at[pl.ds(NT * RPT, RTAIL)])

    plsc.subcore_barrier()

    def chunk(j, carry):
        pltpu.async_copy(ms_hbm.at[srcl.at[j]], buf, sem).wait()
        pltpu.sync_copy(buf, acc.at[dstl.at[j]], add=True)
        return carry

    lax.fori_loop(0, ECH, chunk, 0)
    plsc.subcore_barrier()
    pltpu.sync_copy(acc.at[pl.ds(s * RPT, RPT)], out_hbm.at[pl.ds(s * RPT, RPT)])

    @pl.when(s == 0)
    def _():
        pltpu.sync_copy(acc.at[pl.ds(NT * RPT, RTAIL)],
                        out_hbm.at[pl.ds(NT * RPT, RTAIL)])

    plsc.subcore_barrier()


def _scatter_body(ms0, ms1, ms2, ms3, src_hbm, dst_hbm,
                  out0, out1, out2, out3, acc, srcl, dstl, buf, sem):
    c = lax.axis_index("c")
    s = lax.axis_index("s")
    pltpu.sync_copy(src_hbm.at[s], srcl)
    pltpu.sync_copy(dst_hbm.at[s], dstl)
    mss = (ms0, ms1, ms2, ms3)
    outs = (out0, out1, out2, out3)
    for half in range(2):
        @pl.when(c == 0)
        def _():
            _sc_pass(mss[half], outs[half], acc, srcl, dstl, buf, sem, s)

        @pl.when(c == 1)
        def _():
            _sc_pass(mss[2 + half], outs[2 + half], acc, srcl, dstl,
                     buf, sem, s)


def _make_scatter():
    mesh = plsc.VectorSubcoreMesh(core_axis_name="c", subcore_axis_name="s")
    return pl.kernel(
        _scatter_body,
        out_type=[jax.ShapeDtypeStruct((N, CW), jnp.float32)] * NCH,
        mesh=mesh,
        scratch_types=[
            pltpu.VMEM_SHARED((N, CW), jnp.float32),
            pltpu.VMEM((ECH, EC), jnp.int32),
            pltpu.VMEM((ECH, EC), jnp.int32),
            pltpu.VMEM((EC, CW), jnp.float32),
            pltpu.SemaphoreType.DMA,
        ],
    )


def _h1_body(a0, a1, a2, a3, deg_ref, bg2_ref, wa1_ref, ba1_ref,
             wa2_ref, ba2_ref, bt_ref, adv_ref, sh_ref, sa_ref,
             sh_acc, sa_acc):
    i = pl.program_id(0)
    dinv = 1.0 / jnp.sqrt(deg_ref[...] + 1.0)
    agg = jnp.concatenate([a0[...], a1[...], a2[...], a3[...]], axis=1)
    h = jnp.maximum(agg * dinv + bg2_ref[...], 0.0)
    z1 = jnp.maximum(jnp.dot(h, wa1_ref[...],
                             preferred_element_type=jnp.float32)
                     + ba1_ref[...], 0.0)
    adv = jnp.dot(z1, wa2_ref[...],
                  preferred_element_type=jnp.float32) + ba2_ref[...]
    adv_ref[...] = adv
    ob = (bt_ref[...] == lax.broadcasted_iota(jnp.int32, (BLK, G), 1)
          ).astype(jnp.float32)

    @pl.when(i == 0)
    def _():
        sh_acc[...] = jnp.zeros_like(sh_acc)
        sa_acc[...] = jnp.zeros_like(sa_acc)

    sh_acc[...] += lax.dot_general(ob, h, (((0,), (0,)), ((), ())),
                                   preferred_element_type=jnp.float32)
    adv_cnt = jnp.concatenate(
        [adv, jnp.ones((BLK, 1), jnp.float32)], axis=1)
    sa_acc[...] += lax.dot_general(ob, adv_cnt, (((0,), (0,)), ((), ())),
                                   preferred_element_type=jnp.float32)

    @pl.when(i == pl.num_programs(0) - 1)
    def _():
        sh_ref[...] = sh_acc[...]
        sa_ref[...] = sa_acc[...]


def _heads1(aggs, deg, b_g2, W_a1, b_a1, W_a2, b_a2, bt):
    return pl.pallas_call(
        _h1_body,
        grid=(N // BLK,),
        in_specs=[pl.BlockSpec((BLK, CW), lambda i: (i, 0))] * NCH + [
            pl.BlockSpec((BLK, 1), lambda i: (i, 0)),
            pl.BlockSpec((1, H), lambda i: (0, 0)),
            pl.BlockSpec((H, H), lambda i: (0, 0)),
            pl.BlockSpec((1, H), lambda i: (0, 0)),
            pl.BlockSpec((H, 1), lambda i: (0, 0)),
            pl.BlockSpec((1, 1), lambda i: (0, 0)),
            pl.BlockSpec((BLK, 1), lambda i: (i, 0)),
        ],
        out_specs=[
            pl.BlockSpec((BLK, 1), lambda i: (i, 0)),
            pl.BlockSpec((G, H), lambda i: (0, 0)),
            pl.BlockSpec((G, 2), lambda i: (0, 0)),
        ],
        out_shape=[
            jax.ShapeDtypeStruct((N, 1), jnp.float32),
            jax.ShapeDtypeStruct((G, H), jnp.float32),
            jax.ShapeDtypeStruct((G, 2), jnp.float32),
        ],
        scratch_shapes=[pltpu.VMEM((G, H), jnp.float32),
                        pltpu.VMEM((G, 2), jnp.float32)],
    )(*aggs, deg, b_g2, W_a1, b_a1, W_a2, b_a2, bt)


def _h2_body(sh_ref, sa_ref, wv1_ref, bv1_ref, wv2_ref, bv2_ref,
             adv_ref, bt_ref, q_ref):
    cnt = jnp.maximum(sa_ref[:, 1:2], 1.0)
    vx = sh_ref[...] / cnt
    z1 = jnp.maximum(jnp.dot(vx, wv1_ref[...],
                             preferred_element_type=jnp.float32)
                     + bv1_ref[...], 0.0)
    value = jnp.dot(z1, wv2_ref[...],
                    preferred_element_type=jnp.float32) + bv2_ref[...]
    corr = value - sa_ref[:, 0:1] / cnt
    ob = (bt_ref[...] == lax.broadcasted_iota(jnp.int32, (BLK, G), 1)
          ).astype(jnp.float32)
    q_ref[...] = adv_ref[...] + jnp.dot(ob, corr,
                                        preferred_element_type=jnp.float32)


def _heads2(sh, sa, W_v1, b_v1, W_v2, b_v2, adv, bt):
    return pl.pallas_call(
        _h2_body,
        grid=(N // BLK,),
        in_specs=[
            pl.BlockSpec((G, H), lambda i: (0, 0)),
            pl.BlockSpec((G, 2), lambda i: (0, 0)),
            pl.BlockSpec((H, H), lambda i: (0, 0)),
            pl.BlockSpec((1, H), lambda i: (0, 0)),
            pl.BlockSpec((H, 1), lambda i: (0, 0)),
            pl.BlockSpec((1, 1), lambda i: (0, 0)),
            pl.BlockSpec((BLK, 1), lambda i: (i, 0)),
            pl.BlockSpec((BLK, 1), lambda i: (i, 0)),
        ],
        out_specs=pl.BlockSpec((BLK, 1), lambda i: (i, 0)),
        out_shape=jax.ShapeDtypeStruct((N, 1), jnp.float32),
    )(sh, sa, W_v1, b_v1, W_v2, b_v2, adv, bt)


def kernel(x, edge_index, batch, part_ids, embeddings, W_enc, b_enc,
           W_g0, b_g0, W_g1, b_g1, W_g2, b_g2,
           W_a1, b_a1, W_a2, b_a2, W_v1, b_v1, W_v2, b_v2):
    src0, dst0 = edge_index[0], edge_index[1]
    src_r = src0.reshape(NT, ECH, EC)
    dst_r = dst0.reshape(NT, ECH, EC)

    deg = jax.ops.segment_sum(jnp.ones_like(src0, dtype=jnp.float32), dst0,
                              num_segments=N)[:, None]
    pe = jnp.take(embeddings, part_ids, axis=0)

    h = _encoder(pe, x, W_enc, b_enc[None, :])

    scat = _make_scatter()
    ms = _layer_mm0(h, W_g0, deg)
    aggs = scat(*ms, src_r, dst_r)
    for W, b_prev in ((W_g1, b_g0), (W_g2, b_g1)):
        ms = _layer_mm(aggs, W, deg, b_prev[None, :])
        aggs = scat(*ms, src_r, dst_r)

    bt = batch.astype(jnp.int32)[:, None]
    adv, sh, sa = _heads1(aggs, deg, b_g2[None, :], W_a1, b_a1[None, :],
                          W_a2, b_a2[None, :], bt)
    return _heads2(sh, sa, W_v1, b_v1[None, :], W_v2, b_v2[None, :], adv, bt)
